# baseline (speedup 1.0000x reference)
import functools

import jax
import jax.numpy as jnp
from jax import lax
from jax.experimental import pallas as pl
from jax.experimental.pallas import tpu as pltpu, tpu_sc as plsc

_TOKENS = 8192
_D = 4096
_E = 64
_K = 8
_BLK = 512

_NWORKERS = 32
_RPW = _TOKENS // _NWORKERS
_GRP = 16
_NGRP = _RPW // _GRP


def _probs_kernel(x_ref, wt_ref, b_ref, probs_ref):
    x = x_ref[...]
    wt = wt_ref[...]
    logits = jnp.dot(x, wt, preferred_element_type=jnp.float32) + b_ref[...]
    m = jnp.max(logits, axis=-1, keepdims=True)
    e = jnp.exp(logits - m)
    probs_ref[...] = e / jnp.sum(e, axis=-1, keepdims=True)


def _tc_probs(x, wt, b2):
    grid = (_TOKENS // _BLK,)
    return pl.pallas_call(
        _probs_kernel,
        grid=grid,
        in_specs=[
            pl.BlockSpec((_BLK, _D), lambda i: (i, 0)),
            pl.BlockSpec((_D, _E), lambda i: (0, 0)),
            pl.BlockSpec((1, _E), lambda i: (0, 0)),
        ],
        out_specs=pl.BlockSpec((_BLK, _E), lambda i: (i, 0)),
        out_shape=jax.ShapeDtypeStruct((_TOKENS, _E), jnp.float32),
    )(x, wt, b2)


def _sc_topk_body(probs_hbm, sparse_hbm, idx_hbm, probs_v, sparse_v, idx_v):
    wid = lax.axis_index("s") * 2 + lax.axis_index("c")
    row_base = wid * _RPW

    pltpu.sync_copy(probs_hbm.at[pl.ds(row_base * _E, _RPW * _E)], probs_v)

    zeros16 = jnp.zeros((_GRP,), jnp.float32)

    def _zero_body(i, c):
        base = i * (_GRP * 16)
        for u in range(16):
            sparse_v[pl.ds(base + u * _GRP, _GRP)] = zeros16
        return c

    lax.fori_loop(0, _RPW * _E // (_GRP * 16), _zero_body, 0)

    lane = lax.iota(jnp.int32, _GRP)
    lane64 = lane * _E
    lane8 = lane * _K

    def _group_body(g, c):
        pbase = g * (_GRP * _E)
        work = [plsc.load_gather(probs_v, [lane64 + (pbase + e)])
                for e in range(_E)]

        vals = []
        idxs = []
        for _ in range(_K):
            t = list(work)
            while len(t) > 1:
                t = [jnp.maximum(t[2 * i], t[2 * i + 1])
                     for i in range(len(t) // 2)]
            m = t[0]
            contrib = []
            for e in range(_E):
                hit = work[e] == m
                contrib.append(jnp.where(hit, jnp.int32(e), jnp.int32(0)))
                work[e] = jnp.where(hit, jnp.float32(-1.0), work[e])
            while len(contrib) > 1:
                contrib = [contrib[2 * i] + contrib[2 * i + 1]
                           for i in range(len(contrib) // 2)]
            vals.append(m)
            idxs.append(contrib[0])

        for j in range(_K):
            plsc.store_scatter(sparse_v, [lane64 + pbase + idxs[j]], vals[j])
            plsc.store_scatter(idx_v, [lane8 + (g * _GRP * _K + j)], idxs[j])
        return c

    lax.fori_loop(0, _NGRP, _group_body, 0)

    pltpu.sync_copy(sparse_v, sparse_hbm.at[pl.ds(row_base * _E, _RPW * _E)])
    pltpu.sync_copy(idx_v, idx_hbm.at[pl.ds(row_base * _K, _RPW * _K)])


def _sc_topk(probs_flat):
    mesh = plsc.VectorSubcoreMesh(core_axis_name="c", subcore_axis_name="s")
    run = pl.kernel(
        _sc_topk_body,
        out_type=[
            jax.ShapeDtypeStruct((_TOKENS * _E,), jnp.float32),
            jax.ShapeDtypeStruct((_TOKENS * _K,), jnp.int32),
        ],
        mesh=mesh,
        compiler_params=pltpu.CompilerParams(needs_layout_passes=False),
        scratch_types=[
            pltpu.VMEM((_RPW * _E,), jnp.float32),
            pltpu.VMEM((_RPW * _E,), jnp.float32),
            pltpu.VMEM((_RPW * _K,), jnp.int32),
        ],
    )
    return run(probs_flat)


@jax.jit
def kernel(x, W, b, training):
    del training
    wt = W.T
    b2 = b.reshape(1, _E)
    probs = _tc_probs(x, wt, b2)
    sparse_flat, idx_flat = _sc_topk(probs.reshape(-1))
    return (sparse_flat.reshape(_TOKENS, _E), idx_flat.reshape(_TOKENS, _K))


# speedup vs baseline: 1.7364x; 1.7364x over previous
import jax
import jax.numpy as jnp
from jax.experimental import pallas as pl

_TOKENS = 8192
_D = 4096
_E = 64
_K = 8
_BLK = 512


def _router_kernel(x_ref, wt_ref, b_ref, sparse_ref, idx_ref):
    x = x_ref[...]
    wt = wt_ref[...]
    logits = jnp.dot(x, wt, preferred_element_type=jnp.float32) + b_ref[...]

    m = jnp.max(logits, axis=-1, keepdims=True)
    e = jnp.exp(logits - m)
    probs = e / jnp.sum(e, axis=-1, keepdims=True)

    lane = jax.lax.broadcasted_iota(jnp.int32, probs.shape, 1)
    work = probs
    idx_cols = []
    for _ in range(_K):
        mx = jnp.max(work, axis=-1, keepdims=True)
        hit = work == mx
        arg = jnp.min(jnp.where(hit, lane, _E), axis=-1, keepdims=True)
        idx_cols.append(arg)
        work = jnp.where(lane == arg, -jnp.inf, work)

    sparse_ref[...] = jnp.where(jnp.isneginf(work), probs, 0.0)
    idx_ref[...] = jnp.concatenate(idx_cols, axis=-1)


@jax.jit
def kernel(x, W, b, training):
    del training
    wt = W.T
    b2 = b.reshape(1, _E)
    grid = (_TOKENS // _BLK,)
    sparse, idx = pl.pallas_call(
        _router_kernel,
        grid=grid,
        in_specs=[
            pl.BlockSpec((_BLK, _D), lambda i: (i, 0)),
            pl.BlockSpec((_D, _E), lambda i: (0, 0)),
            pl.BlockSpec((1, _E), lambda i: (0, 0)),
        ],
        out_specs=[
            pl.BlockSpec((_BLK, _E), lambda i: (i, 0)),
            pl.BlockSpec((_BLK, _K), lambda i: (i, 0)),
        ],
        out_shape=[
            jax.ShapeDtypeStruct((_TOKENS, _E), jnp.float32),
            jax.ShapeDtypeStruct((_TOKENS, _K), jnp.int32),
        ],
    )(x, wt, b2)
    return (sparse, idx)


# speedup vs baseline: 2.1267x; 1.2248x over previous
import jax
import jax.numpy as jnp
from jax.experimental import pallas as pl

_TOKENS = 8192
_D = 4096
_E = 64
_K = 8
_BLK = 512


def _router_kernel(x_ref, wt_ref, b_ref, sparse_ref, idx_ref):
    x = x_ref[...]
    wt = wt_ref[...]
    logits = jnp.dot(x, wt, preferred_element_type=jnp.float32) + b_ref[...]

    m = jnp.max(logits, axis=-1, keepdims=True)
    e = jnp.exp(logits - m)
    probs = e / jnp.sum(e, axis=-1, keepdims=True)

    work = probs.T
    lane = jax.lax.broadcasted_iota(jnp.int32, work.shape, 0)
    idx_rows = []
    for _ in range(_K):
        mx = jnp.max(work, axis=0, keepdims=True)
        hit = work == mx
        arg = jnp.min(jnp.where(hit, lane, _E), axis=0, keepdims=True)
        idx_rows.append(arg)
        work = jnp.where(lane == arg, -jnp.inf, work)

    sparse_ref[...] = jnp.where(jnp.isneginf(work.T), probs, 0.0)
    idx_ref[...] = jnp.concatenate(idx_rows, axis=0).T


@jax.jit
def kernel(x, W, b, training):
    del training
    wt = W.T
    b2 = b.reshape(1, _E)
    grid = (_TOKENS // _BLK,)
    sparse, idx = pl.pallas_call(
        _router_kernel,
        grid=grid,
        in_specs=[
            pl.BlockSpec((_BLK, _D), lambda i: (i, 0)),
            pl.BlockSpec((_D, _E), lambda i: (0, 0)),
            pl.BlockSpec((1, _E), lambda i: (0, 0)),
        ],
        out_specs=[
            pl.BlockSpec((_BLK, _E), lambda i: (i, 0)),
            pl.BlockSpec((_BLK, _K), lambda i: (i, 0)),
        ],
        out_shape=[
            jax.ShapeDtypeStruct((_TOKENS, _E), jnp.float32),
            jax.ShapeDtypeStruct((_TOKENS, _K), jnp.int32),
        ],
    )(x, wt, b2)
    return (sparse, idx)
